# baseline (speedup 1.0000x reference)
import jax
import jax.numpy as jnp
from jax.experimental import pallas as pl


def _vq_block_kernel(z_ref, embt_ref, out_ref):
    z = z_ref[0]
    embt = embt_ref[...]
    c, k = embt.shape

    e_sq = jnp.sum(embt * embt, axis=0).reshape(k, 1)
    scores = jax.lax.dot_general(
        embt, z, (((0,), (0,)), ((), ())),
        preferred_element_type=jnp.float32)
    d2 = e_sq - 2.0 * scores

    minv = jnp.min(d2, axis=0, keepdims=True)
    rows = jax.lax.broadcasted_iota(jnp.int32, d2.shape, 0)
    idx = jnp.min(jnp.where(d2 == minv, rows, k), axis=0)

    t = idx.shape[0]
    sub_b = jnp.broadcast_to((idx & 127)[None, :], (c, t))
    hi = idx >> 7
    acc = jnp.zeros((c, t), jnp.float32)
    for j in range(k // 128):
        g = jnp.take_along_axis(embt[:, j * 128:(j + 1) * 128], sub_b,
                                axis=1)
        acc = jnp.where((hi == j)[None, :], g, acc)
    out_ref[0] = acc


def kernel(z_e, embedding):
    B, C, H, W = z_e.shape
    K = embedding.shape[0]
    T = H * W
    z = z_e.reshape(B, C, T)
    embt = embedding.T
    out = pl.pallas_call(
        _vq_block_kernel,
        grid=(B,),
        in_specs=[
            pl.BlockSpec((1, C, T), lambda b: (b, 0, 0)),
            pl.BlockSpec((C, K), lambda b: (0, 0)),
        ],
        out_specs=pl.BlockSpec((1, C, T), lambda b: (b, 0, 0)),
        out_shape=jax.ShapeDtypeStruct((B, C, T), jnp.float32),
    )(z, embt)
    return out.reshape(B, C, H, W)


# speedup vs baseline: 1.2080x; 1.2080x over previous
import jax
import jax.numpy as jnp
from jax.experimental import pallas as pl


def _vq_block_kernel(z_ref, emb_ref, embn2_ref, out_ref):
    z = z_ref[0]
    emb = emb_ref[...]
    embn2 = embn2_ref[...]
    k = emb.shape[0]

    e_sq = jnp.sum(emb * emb, axis=1, keepdims=True)
    scores = jax.lax.dot_general(
        embn2, z, (((1,), (0,)), ((), ())),
        preferred_element_type=jnp.float32)
    d2 = scores + e_sq

    minv = jnp.min(d2, axis=0, keepdims=True)
    rows = jax.lax.broadcasted_iota(jnp.int32, d2.shape, 0)
    idx = jnp.min(jnp.where(d2 == minv, rows, k), axis=0, keepdims=True)

    onehot = (rows == idx).astype(jnp.float32)
    out_ref[0] = jax.lax.dot_general(
        emb, onehot, (((0,), (0,)), ((), ())),
        preferred_element_type=jnp.float32)


def kernel(z_e, embedding):
    B, C, H, W = z_e.shape
    K = embedding.shape[0]
    T = H * W
    z = z_e.reshape(B, C, T)
    out = pl.pallas_call(
        _vq_block_kernel,
        grid=(B,),
        in_specs=[
            pl.BlockSpec((1, C, T), lambda b: (b, 0, 0)),
            pl.BlockSpec((K, C), lambda b: (0, 0)),
            pl.BlockSpec((K, C), lambda b: (0, 0)),
        ],
        out_specs=pl.BlockSpec((1, C, T), lambda b: (b, 0, 0)),
        out_shape=jax.ShapeDtypeStruct((B, C, T), jnp.float32),
    )(z, embedding, -2.0 * embedding)
    return out.reshape(B, C, H, W)
